# baseline (speedup 1.0000x reference)
import functools

import jax
import jax.numpy as jnp
from jax import lax
from jax.experimental import pallas as pl
from jax.experimental.pallas import tpu as pltpu
from jax.experimental.pallas import tpu_sc as plsc

_LANES = 16
_SCALE = 8.0
_NBUF = 4


def _emb_call(B, V, D, NW, CH, n_chunks):
    mesh = plsc.VectorSubcoreMesh(core_axis_name="c", subcore_axis_name="s")
    num_cores = mesh.num_cores
    K = n_chunks // _NBUF

    @functools.partial(
        pl.kernel,
        out_type=jax.ShapeDtypeStruct((B, D), jnp.float32),
        mesh=mesh,
        scratch_types=[
            pltpu.VMEM((n_chunks, CH), jnp.int32),
            [pltpu.VMEM((CH, D), jnp.float32) for _ in range(_NBUF)],
            [pltpu.VMEM((CH, D), jnp.float32) for _ in range(_NBUF)],
            [pltpu.SemaphoreType.DMA for _ in range(_NBUF)],
            [pltpu.SemaphoreType.DMA for _ in range(_NBUF)],
        ],
        compiler_params=pltpu.CompilerParams(use_tc_tiling_on_sc=False),
    )
    def emb_kernel(idx_hbm, table_hbm, out_hbm, idx_v, rows_g, rows_s, gsem, ssem):
        wid = lax.axis_index("s") * num_cores + lax.axis_index("c")
        out_base = wid * n_chunks * CH
        pltpu.sync_copy(idx_hbm.at[wid], idx_v)

        def scale(b):
            @pl.loop(0, CH, unroll=8)
            def _row(i):
                for d in range(D // _LANES):
                    sl = pl.ds(d * _LANES, _LANES)
                    rows_s[b][i, sl] = rows_g[b][i, sl] * _SCALE

        def visit(j, b, first, last):
            pltpu.make_async_copy(
                table_hbm.at[idx_v.at[j]], rows_g[b], gsem[b]
            ).wait()
            if not first:
                pltpu.make_async_copy(
                    rows_s[b],
                    out_hbm.at[pl.ds(out_base + (j - _NBUF) * CH, CH)],
                    ssem[b],
                ).wait()
            scale(b)
            pltpu.async_copy(
                rows_s[b], out_hbm.at[pl.ds(out_base + j * CH, CH)], ssem[b]
            )
            if not last:
                pltpu.async_copy(
                    table_hbm.at[idx_v.at[j + _NBUF]], rows_g[b], gsem[b]
                )

        for b in range(_NBUF):
            pltpu.async_copy(table_hbm.at[idx_v.at[b]], rows_g[b], gsem[b])
        for b in range(_NBUF):
            visit(b, b, first=True, last=False)
        @pl.loop(1, K - 1)
        def _block(k):
            for b in range(_NBUF):
                visit(k * _NBUF + b, b, first=False, last=False)
        for b in range(_NBUF):
            visit((K - 1) * _NBUF + b, b, first=False, last=True)
        for b in range(_NBUF):
            pltpu.make_async_copy(
                rows_s[b],
                out_hbm.at[pl.ds(out_base + ((K - 1) * _NBUF + b) * CH, CH)],
                ssem[b],
            ).wait()

    return emb_kernel


def kernel(inputs, table):
    B0, B1 = inputs.shape
    V, D = table.shape
    B = B0 * B1
    NW = 32
    CH = 128
    n_chunks = B // (NW * CH)

    idx = inputs.reshape(NW, n_chunks, CH).astype(jnp.int32)
    out = _emb_call(B, V, D, NW, CH, n_chunks)(idx, table)
    return out.reshape(B0, B1, D)
